# baseline (speedup 1.0000x reference)
import functools

import jax
import jax.numpy as jnp
from jax import lax
from jax.experimental import pallas as pl
from jax.experimental.pallas import tpu as pltpu
from jax.experimental.pallas import tpu_sc as plsc

DHAT = 0.05
DHAT2 = DHAT * DHAT
INV_DHAT2 = 1.0 / DHAT2

N_CORES = 2
N_SUBCORES = 16
N_TILES = N_CORES * N_SUBCORES
LANES = 16

BLK = 128
NB0 = 31
NB1 = 69
CPT0 = NB0 * BLK
CPT1 = NB1 * BLK
PART0 = N_SUBCORES * CPT0
TOTAL_C = N_SUBCORES * (CPT0 + CPT1)
CPT_MAX = CPT1

TAB_SLICE = 3136
TAB_PAD = N_SUBCORES * TAB_SLICE
U_SUB = TAB_SLICE // 4


def _ln(x):
    xi = plsc.bitcast(x, jnp.int32)
    e = ((xi >> 23) & 0xFF) - 127
    m = plsc.bitcast((xi & 0x7FFFFF) | 0x3F800000, jnp.float32)
    big = m > 1.4142135
    m = jnp.where(big, m * 0.5, m)
    ef = e.astype(jnp.float32) + jnp.where(big, 1.0, 0.0)
    t = (m - 1.0) / (m + 1.0)
    t2 = t * t
    p = t * (2.0 + t2 * (0.6666667 + t2 * (0.4 + t2 * (0.2857143 + t2 * 0.22222223))))
    return ef * 0.6931472 + p


def _make_sc_kernel(n_cands):
    mesh = plsc.VectorSubcoreMesh(core_axis_name="c", subcore_axis_name="s")

    @functools.partial(
        pl.kernel,
        out_type=jax.ShapeDtypeStruct((N_TILES, LANES), jnp.float32),
        mesh=mesh,
        scratch_types=[
            pltpu.VMEM((TAB_PAD,), jnp.float32),
            pltpu.VMEM((TAB_PAD,), jnp.float32),
            pltpu.VMEM((U_SUB,), jnp.float32),
            pltpu.VMEM((CPT_MAX // 2,), jnp.int32),
            pltpu.VMEM((CPT_MAX,), jnp.int32),
            pltpu.VMEM((CPT_MAX,), jnp.int32),
            pltpu.VMEM((LANES,), jnp.float32),
            pltpu.VMEM_SHARED((TAB_PAD,), jnp.float32),
            pltpu.VMEM_SHARED((TAB_PAD,), jnp.float32),
            pltpu.SemaphoreType.DMA,
            pltpu.SemaphoreType.DMA,
        ],
        compiler_params=pltpu.CompilerParams(
            needs_layout_passes=False, use_tc_tiling_on_sc=False),
    )
    def sck(restx_hbm, resty_hbm, ux_hbm, uy_hbm, epk_hbm, cvp_hbm, ce_hbm,
            out_hbm, currx_v, curry_v, ubuf_v, cv_v, ce_v, epk_v, acc_v,
            currx_sh, curry_sh, sem_s, sem_g):
        c = lax.axis_index("c")
        s = lax.axis_index("s")
        wid = s * N_CORES + c

        nb = jnp.where(c == 1, NB0, NB1)
        cand0 = pl.multiple_of(
            jnp.where(c == 1, s * CPT0, PART0 + s * CPT1), BLK)
        word0 = pl.multiple_of(cand0 // 2, BLK // 2)
        halfw = pl.multiple_of(
            jnp.where(c == 1, CPT0 // 2, CPT1 // 2), LANES)

        with jax.named_scope("stage_idx"):
            @pl.loop(0, nb)
            def _(k):
                pltpu.make_async_copy(
                    cvp_hbm.at[pl.ds(word0 + k * (BLK // 2), BLK // 2)],
                    cv_v.at[pl.ds(k * (BLK // 2), BLK // 2)], sem_s).start()
                pltpu.make_async_copy(
                    ce_hbm.at[pl.ds(cand0 + k * BLK, BLK)],
                    ce_v.at[pl.ds(k * BLK, BLK)], sem_s).start()

            @pl.loop(0, nb)
            def _(k):
                pltpu.make_async_copy(
                    cvp_hbm.at[pl.ds(word0 + k * (BLK // 2), BLK // 2)],
                    cv_v.at[pl.ds(k * (BLK // 2), BLK // 2)], sem_s).wait()
                pltpu.make_async_copy(
                    ce_hbm.at[pl.ds(cand0 + k * BLK, BLK)],
                    ce_v.at[pl.ds(k * BLK, BLK)], sem_s).wait()

        with jax.named_scope("fire_gathers"):
            @pl.loop(0, nb)
            def _(k):
                pltpu.make_async_copy(
                    epk_hbm.at[ce_v.at[pl.ds(k * BLK, BLK)]],
                    epk_v.at[pl.ds(k * BLK, BLK)], sem_g).start()

        myoff = s * TAB_SLICE
        sl = pl.ds(myoff, TAB_SLICE)
        with jax.named_scope("build_slices"):
            for rtab, utab, tab_v, tab_sh in (
                    (restx_hbm, ux_hbm, currx_v, currx_sh),
                    (resty_hbm, uy_hbm, curry_v, curry_sh)):
                pltpu.sync_copy(rtab.at[sl], tab_v.at[sl])

                for q in range(TAB_SLICE // U_SUB):
                    pltpu.sync_copy(utab.at[pl.ds(myoff + q * U_SUB, U_SUB)],
                                    ubuf_v)

                    @pl.loop(0, U_SUB // LANES)
                    def _(j, tab_v=tab_v, q=q):
                        d = pl.ds(myoff + q * U_SUB + j * LANES, LANES)
                        tab_v[d] = tab_v[d] + ubuf_v[pl.ds(j * LANES, LANES)]

                pltpu.sync_copy(tab_v.at[sl], tab_sh.at[sl])

        with jax.named_scope("barrier"):
            plsc.subcore_barrier()
        with jax.named_scope("broadcast"):
            pltpu.sync_copy(currx_sh, currx_v)
            pltpu.sync_copy(curry_sh, curry_v)

        with jax.named_scope("drain_gathers"):
            @pl.loop(0, nb)
            def _(k):
                pltpu.make_async_copy(
                    epk_hbm.at[ce_v.at[pl.ds(k * BLK, BLK)]],
                    epk_v.at[pl.ds(k * BLK, BLK)], sem_g).wait()

        lanes = lax.iota(jnp.int32, LANES)

        def contrib(cv, pk, g):
            e0i = pk & 0xFFFF
            e1i = (pk >> 16) & 0xFFFF
            px = plsc.load_gather(currx_v, [cv])
            py = plsc.load_gather(curry_v, [cv])
            e0x = plsc.load_gather(currx_v, [e0i])
            e0y = plsc.load_gather(curry_v, [e0i])
            e1x = plsc.load_gather(currx_v, [e1i])
            e1y = plsc.load_gather(curry_v, [e1i])

            dex = e1x - e0x
            dey = e1y - e0y
            dd = jnp.maximum(dex * dex + dey * dey, 1e-12)
            qx = px - e0x
            qy = py - e0y
            t = (qx * dex + qy * dey) / dd
            t = jnp.minimum(jnp.maximum(t, 0.0), 1.0)
            cx = e0x + t * dex
            cy = e0y + t * dey
            dx = px - cx
            dy = py - cy
            d2 = dx * dx + dy * dy

            active = (d2 < DHAT2) & (d2 > 0.0)
            d2s = jnp.where(active, d2, DHAT2)
            diff = d2s - DHAT2
            b = -(diff * diff) * _ln(d2s * INV_DHAT2)
            return jnp.where(active & (g < n_cands), b, 0.0)

        with jax.named_scope("mainloop"):
            @plsc.parallel_loop(0, nb * (BLK // (2 * LANES)),
                                carry=jnp.zeros((LANES,), jnp.float32))
            def acc(j, acc_in):
                jb = j * LANES
                cvp = cv_v[pl.ds(jb, LANES)]
                pk_lo = epk_v[pl.ds(jb, LANES)]
                pk_hi = epk_v[pl.ds(halfw + jb, LANES)]
                b_lo = contrib(cvp & 0xFFFF, pk_lo, cand0 + jb + lanes)
                b_hi = contrib((cvp >> 16) & 0xFFFF, pk_hi,
                               cand0 + halfw + jb + lanes)
                return acc_in + b_lo + b_hi

            acc_v[...] = acc
        pltpu.sync_copy(acc_v, out_hbm.at[wid])

    return sck


def kernel(U, rest, edges, cand_v, cand_e):
    n_verts = rest.shape[0]
    n_cands = cand_v.shape[0]
    pad_v = TAB_PAD - n_verts
    restx = jnp.pad(rest[:, 0], (0, pad_v))
    resty = jnp.pad(rest[:, 1], (0, pad_v))
    ux = jnp.pad(U[:, 0], (0, pad_v))
    uy = jnp.pad(U[:, 1], (0, pad_v))
    epk = edges[:, 0] | (edges[:, 1] << 16)
    pad_c = TOTAL_C - n_cands
    cv_pad = jnp.pad(cand_v, (0, pad_c))
    ce = jnp.pad(cand_e, (0, pad_c))
    p0 = cv_pad[:PART0].reshape(N_SUBCORES, 2, CPT0 // 2)
    p1 = cv_pad[PART0:].reshape(N_SUBCORES, 2, CPT1 // 2)
    cvp = jnp.concatenate([(p0[:, 0] | (p0[:, 1] << 16)).reshape(-1),
                           (p1[:, 0] | (p1[:, 1] << 16)).reshape(-1)])
    out = _make_sc_kernel(n_cands)(restx, resty, ux, uy, epk, cvp, ce)
    return jnp.sum(out)


# speedup vs baseline: 1.1374x; 1.1374x over previous
import functools

import jax
import jax.numpy as jnp
from jax import lax
from jax.experimental import pallas as pl
from jax.experimental.pallas import tpu as pltpu
from jax.experimental.pallas import tpu_sc as plsc

DHAT = 0.05
DHAT2 = DHAT * DHAT
INV_DHAT2 = 1.0 / DHAT2

N_CORES = 2
N_SUBCORES = 16
N_TILES = N_CORES * N_SUBCORES
LANES = 16

CAND_PER_TILE = 6400
GATHER_W = 128
N_GATHERS = CAND_PER_TILE // GATHER_W
N_ITERS = CAND_PER_TILE // (2 * LANES)
HALF = CAND_PER_TILE // 2

TAB_SLICE = 3136
TAB_PAD = N_SUBCORES * TAB_SLICE


def _ln(x):
    xi = plsc.bitcast(x, jnp.int32)
    e = ((xi >> 23) & 0xFF) - 127
    m = plsc.bitcast((xi & 0x7FFFFF) | 0x3F800000, jnp.float32)
    big = m > 1.4142135
    m = jnp.where(big, m * 0.5, m)
    ef = e.astype(jnp.float32) + jnp.where(big, 1.0, 0.0)
    t = (m - 1.0) / (m + 1.0)
    t2 = t * t
    p = t * (2.0 + t2 * (0.6666667 + t2 * (0.4 + t2 * (0.2857143 + t2 * 0.22222223))))
    return ef * 0.6931472 + p


def _make_sc_kernel(n_cands):
    mesh = plsc.VectorSubcoreMesh(core_axis_name="c", subcore_axis_name="s")

    @functools.partial(
        pl.kernel,
        out_type=jax.ShapeDtypeStruct((N_TILES, LANES), jnp.float32),
        mesh=mesh,
        scratch_types=[
            pltpu.VMEM((TAB_PAD,), jnp.float32),
            pltpu.VMEM((TAB_PAD,), jnp.float32),
            pltpu.VMEM((TAB_SLICE,), jnp.float32),
            pltpu.VMEM((TAB_SLICE,), jnp.float32),
            pltpu.VMEM((HALF,), jnp.int32),
            pltpu.VMEM((CAND_PER_TILE,), jnp.int32),
            pltpu.VMEM((CAND_PER_TILE,), jnp.int32),
            pltpu.VMEM((LANES,), jnp.float32),
            pltpu.VMEM_SHARED((TAB_PAD,), jnp.float32),
            pltpu.VMEM_SHARED((TAB_PAD,), jnp.float32),
            pltpu.SemaphoreType.DMA,
        ],
        compiler_params=pltpu.CompilerParams(
            needs_layout_passes=False, use_tc_tiling_on_sc=False),
    )
    def sck(restx_hbm, resty_hbm, ux_hbm, uy_hbm, epk_hbm, cvp_hbm, ce_hbm,
            out_hbm, currx_v, curry_v, ubufx_v, ubufy_v, cv_v, ce_v, epk_v,
            acc_v, currx_sh, curry_sh, sem_g):
        c = lax.axis_index("c")
        s = lax.axis_index("s")
        wid = s * N_CORES + c
        myoff = s * TAB_SLICE
        sl = pl.ds(myoff, TAB_SLICE)

        with jax.named_scope("stage"):
            pltpu.sync_copy(cvp_hbm.at[wid], cv_v)
            pltpu.sync_copy(ce_hbm.at[wid], ce_v)
            pltpu.sync_copy(restx_hbm.at[sl], currx_v.at[sl])
            pltpu.sync_copy(resty_hbm.at[sl], curry_v.at[sl])
            pltpu.sync_copy(ux_hbm.at[sl], ubufx_v)
            pltpu.sync_copy(uy_hbm.at[sl], ubufy_v)

        with jax.named_scope("fire_gathers"):
            @pl.loop(0, N_GATHERS)
            def _(k):
                pltpu.make_async_copy(
                    epk_hbm.at[ce_v.at[pl.ds(k * GATHER_W, GATHER_W)]],
                    epk_v.at[pl.ds(k * GATHER_W, GATHER_W)], sem_g).start()

        with jax.named_scope("build_slices"):
            @pl.loop(0, TAB_SLICE // LANES)
            def _(j):
                d = pl.ds(myoff + j * LANES, LANES)
                b = pl.ds(j * LANES, LANES)
                currx_v[d] = currx_v[d] + ubufx_v[b]
                curry_v[d] = curry_v[d] + ubufy_v[b]

            pltpu.sync_copy(currx_v.at[sl], currx_sh.at[sl])
            pltpu.sync_copy(curry_v.at[sl], curry_sh.at[sl])

        with jax.named_scope("barrier"):
            plsc.subcore_barrier()
        with jax.named_scope("broadcast"):
            pltpu.sync_copy(currx_sh, currx_v)
            pltpu.sync_copy(curry_sh, curry_v)

        with jax.named_scope("drain_gathers"):
            @pl.loop(0, N_GATHERS)
            def _(k):
                pltpu.make_async_copy(
                    epk_hbm.at[ce_v.at[pl.ds(k * GATHER_W, GATHER_W)]],
                    epk_v.at[pl.ds(k * GATHER_W, GATHER_W)], sem_g).wait()

        lanes = lax.iota(jnp.int32, LANES)
        base_g = wid * CAND_PER_TILE

        def contrib(cv, pk, g):
            e0i = pk & 0xFFFF
            e1i = (pk >> 16) & 0xFFFF
            px = plsc.load_gather(currx_v, [cv])
            py = plsc.load_gather(curry_v, [cv])
            e0x = plsc.load_gather(currx_v, [e0i])
            e0y = plsc.load_gather(curry_v, [e0i])
            e1x = plsc.load_gather(currx_v, [e1i])
            e1y = plsc.load_gather(curry_v, [e1i])

            dex = e1x - e0x
            dey = e1y - e0y
            dd = jnp.maximum(dex * dex + dey * dey, 1e-12)
            qx = px - e0x
            qy = py - e0y
            t = (qx * dex + qy * dey) / dd
            t = jnp.minimum(jnp.maximum(t, 0.0), 1.0)
            cx = e0x + t * dex
            cy = e0y + t * dey
            dx = px - cx
            dy = py - cy
            d2 = dx * dx + dy * dy

            active = (d2 < DHAT2) & (d2 > 0.0)
            d2s = jnp.where(active, d2, DHAT2)
            diff = d2s - DHAT2
            b = -(diff * diff) * _ln(d2s * INV_DHAT2)
            return jnp.where(active & (g < n_cands), b, 0.0)

        with jax.named_scope("mainloop"):
            @plsc.parallel_loop(0, N_ITERS,
                                carry=jnp.zeros((LANES,), jnp.float32))
            def acc(j, acc_in):
                jb = j * LANES
                cvp = cv_v[pl.ds(jb, LANES)]
                pk_lo = epk_v[pl.ds(jb, LANES)]
                pk_hi = epk_v[pl.ds(HALF + jb, LANES)]
                b_lo = contrib(cvp & 0xFFFF, pk_lo, base_g + jb + lanes)
                b_hi = contrib((cvp >> 16) & 0xFFFF, pk_hi,
                               base_g + HALF + jb + lanes)
                return acc_in + b_lo + b_hi

            acc_v[...] = acc
        pltpu.sync_copy(acc_v, out_hbm.at[wid])

    return sck


def kernel(U, rest, edges, cand_v, cand_e):
    n_verts = rest.shape[0]
    n_cands = cand_v.shape[0]
    pad_v = TAB_PAD - n_verts
    restx = jnp.pad(rest[:, 0], (0, pad_v))
    resty = jnp.pad(rest[:, 1], (0, pad_v))
    ux = jnp.pad(U[:, 0], (0, pad_v))
    uy = jnp.pad(U[:, 1], (0, pad_v))
    epk = edges[:, 0] | (edges[:, 1] << 16)
    pad_c = N_TILES * CAND_PER_TILE - n_cands
    cv2 = jnp.pad(cand_v, (0, pad_c)).reshape(N_TILES, 2, HALF)
    cvp = cv2[:, 0] | (cv2[:, 1] << 16)
    ce = jnp.pad(cand_e, (0, pad_c)).reshape(N_TILES, CAND_PER_TILE)
    out = _make_sc_kernel(n_cands)(restx, resty, ux, uy, epk, cvp, ce)
    return jnp.sum(out)


# speedup vs baseline: 1.1830x; 1.0401x over previous
import functools

import jax
import jax.numpy as jnp
from jax import lax
from jax.experimental import pallas as pl
from jax.experimental.pallas import tpu as pltpu
from jax.experimental.pallas import tpu_sc as plsc

DHAT = 0.05
DHAT2 = DHAT * DHAT
INV_DHAT2 = 1.0 / DHAT2

N_CORES = 2
N_SUBCORES = 16
N_TILES = N_CORES * N_SUBCORES
LANES = 16

CAND_PER_TILE = 6400
GATHER_W = 128
N_GATHERS = CAND_PER_TILE // GATHER_W
N_ITERS = CAND_PER_TILE // (2 * LANES)
HALF = CAND_PER_TILE // 2

TAB_SLICE = 3136
TAB_PAD = N_SUBCORES * TAB_SLICE


def _ln(x):
    xi = plsc.bitcast(x, jnp.int32)
    e = ((xi >> 23) & 0xFF) - 127
    m = plsc.bitcast((xi & 0x7FFFFF) | 0x3F800000, jnp.float32)
    big = m > 1.4142135
    m = jnp.where(big, m * 0.5, m)
    ef = e.astype(jnp.float32) + jnp.where(big, 1.0, 0.0)
    t = (m - 1.0) / (m + 1.0)
    t2 = t * t
    p = t * (2.0 + t2 * (0.6666667 + t2 * (0.4 + t2 * (0.2857143 + t2 * 0.22222223))))
    return ef * 0.6931472 + p


def _make_sc_kernel(n_cands):
    mesh = plsc.VectorSubcoreMesh(core_axis_name="c", subcore_axis_name="s")

    @functools.partial(
        pl.kernel,
        out_type=jax.ShapeDtypeStruct((N_TILES, LANES), jnp.float32),
        mesh=mesh,
        scratch_types=[
            pltpu.VMEM((TAB_PAD,), jnp.float32),
            pltpu.VMEM((TAB_PAD,), jnp.float32),
            pltpu.VMEM((TAB_SLICE,), jnp.float32),
            pltpu.VMEM((TAB_SLICE,), jnp.float32),
            pltpu.VMEM((HALF,), jnp.int32),
            pltpu.VMEM((CAND_PER_TILE,), jnp.int32),
            pltpu.VMEM((CAND_PER_TILE,), jnp.int32),
            pltpu.VMEM((LANES,), jnp.float32),
            pltpu.VMEM_SHARED((TAB_PAD,), jnp.float32),
            pltpu.VMEM_SHARED((TAB_PAD,), jnp.float32),
            pltpu.SemaphoreType.DMA,
            pltpu.SemaphoreType.DMA,
        ],
        compiler_params=pltpu.CompilerParams(
            needs_layout_passes=False, use_tc_tiling_on_sc=False),
    )
    def sck(restx_hbm, resty_hbm, ux_hbm, uy_hbm, epk_hbm, cvp_hbm, ce_hbm,
            out_hbm, currx_v, curry_v, ubufx_v, ubufy_v, cv_v, ce_v, epk_v,
            acc_v, currx_sh, curry_sh, sem_g, sem_s):
        c = lax.axis_index("c")
        s = lax.axis_index("s")
        wid = s * N_CORES + c
        myoff = s * TAB_SLICE
        sl = pl.ds(myoff, TAB_SLICE)

        with jax.named_scope("stage"):
            stage = [
                pltpu.make_async_copy(cvp_hbm.at[wid], cv_v, sem_s),
                pltpu.make_async_copy(ce_hbm.at[wid], ce_v, sem_s),
                pltpu.make_async_copy(restx_hbm.at[sl], currx_v.at[sl], sem_s),
                pltpu.make_async_copy(resty_hbm.at[sl], curry_v.at[sl], sem_s),
                pltpu.make_async_copy(ux_hbm.at[sl], ubufx_v, sem_s),
                pltpu.make_async_copy(uy_hbm.at[sl], ubufy_v, sem_s),
            ]
            for d in stage:
                d.start()
            for d in stage:
                d.wait()

        with jax.named_scope("fire_gathers"):
            @pl.loop(0, N_GATHERS)
            def _(k):
                pltpu.make_async_copy(
                    epk_hbm.at[ce_v.at[pl.ds(k * GATHER_W, GATHER_W)]],
                    epk_v.at[pl.ds(k * GATHER_W, GATHER_W)], sem_g).start()

        with jax.named_scope("build_slices"):
            @pl.loop(0, TAB_SLICE // LANES)
            def _(j):
                d = pl.ds(myoff + j * LANES, LANES)
                b = pl.ds(j * LANES, LANES)
                currx_v[d] = currx_v[d] + ubufx_v[b]
                curry_v[d] = curry_v[d] + ubufy_v[b]

            pltpu.sync_copy(currx_v.at[sl], currx_sh.at[sl])
            pltpu.sync_copy(curry_v.at[sl], curry_sh.at[sl])

        with jax.named_scope("barrier"):
            plsc.subcore_barrier()
        with jax.named_scope("broadcast"):
            pltpu.sync_copy(currx_sh, currx_v)
            pltpu.sync_copy(curry_sh, curry_v)

        with jax.named_scope("drain_gathers"):
            @pl.loop(0, N_GATHERS)
            def _(k):
                pltpu.make_async_copy(
                    epk_hbm.at[ce_v.at[pl.ds(k * GATHER_W, GATHER_W)]],
                    epk_v.at[pl.ds(k * GATHER_W, GATHER_W)], sem_g).wait()

        lanes = lax.iota(jnp.int32, LANES)
        base_g = wid * CAND_PER_TILE

        def contrib(cv, pk, g):
            e0i = pk & 0xFFFF
            e1i = (pk >> 16) & 0xFFFF
            px = plsc.load_gather(currx_v, [cv])
            py = plsc.load_gather(curry_v, [cv])
            e0x = plsc.load_gather(currx_v, [e0i])
            e0y = plsc.load_gather(curry_v, [e0i])
            e1x = plsc.load_gather(currx_v, [e1i])
            e1y = plsc.load_gather(curry_v, [e1i])

            dex = e1x - e0x
            dey = e1y - e0y
            dd = jnp.maximum(dex * dex + dey * dey, 1e-12)
            qx = px - e0x
            qy = py - e0y
            t = (qx * dex + qy * dey) / dd
            t = jnp.minimum(jnp.maximum(t, 0.0), 1.0)
            cx = e0x + t * dex
            cy = e0y + t * dey
            dx = px - cx
            dy = py - cy
            d2 = dx * dx + dy * dy

            active = (d2 < DHAT2) & (d2 > 0.0)
            d2s = jnp.where(active, d2, DHAT2)
            diff = d2s - DHAT2
            b = -(diff * diff) * _ln(d2s * INV_DHAT2)
            return jnp.where(active & (g < n_cands), b, 0.0)

        with jax.named_scope("mainloop"):
            @plsc.parallel_loop(0, N_ITERS,
                                carry=jnp.zeros((LANES,), jnp.float32))
            def acc(j, acc_in):
                jb = j * LANES
                cvp = cv_v[pl.ds(jb, LANES)]
                pk_lo = epk_v[pl.ds(jb, LANES)]
                pk_hi = epk_v[pl.ds(HALF + jb, LANES)]
                b_lo = contrib(cvp & 0xFFFF, pk_lo, base_g + jb + lanes)
                b_hi = contrib((cvp >> 16) & 0xFFFF, pk_hi,
                               base_g + HALF + jb + lanes)
                return acc_in + b_lo + b_hi

            acc_v[...] = acc
        pltpu.sync_copy(acc_v, out_hbm.at[wid])

    return sck


def kernel(U, rest, edges, cand_v, cand_e):
    n_verts = rest.shape[0]
    n_cands = cand_v.shape[0]
    pad_v = TAB_PAD - n_verts
    restx = jnp.pad(rest[:, 0], (0, pad_v))
    resty = jnp.pad(rest[:, 1], (0, pad_v))
    ux = jnp.pad(U[:, 0], (0, pad_v))
    uy = jnp.pad(U[:, 1], (0, pad_v))
    epk = edges[:, 0] | (edges[:, 1] << 16)
    pad_c = N_TILES * CAND_PER_TILE - n_cands
    cv2 = jnp.pad(cand_v, (0, pad_c)).reshape(N_TILES, 2, HALF)
    cvp = cv2[:, 0] | (cv2[:, 1] << 16)
    ce = jnp.pad(cand_e, (0, pad_c)).reshape(N_TILES, CAND_PER_TILE)
    out = _make_sc_kernel(n_cands)(restx, resty, ux, uy, epk, cvp, ce)
    return jnp.sum(out)


# speedup vs baseline: 1.1851x; 1.0018x over previous
import functools

import jax
import jax.numpy as jnp
from jax import lax
from jax.experimental import pallas as pl
from jax.experimental.pallas import tpu as pltpu
from jax.experimental.pallas import tpu_sc as plsc

DHAT = 0.05
DHAT2 = DHAT * DHAT
INV_DHAT2 = 1.0 / DHAT2

N_CORES = 2
N_SUBCORES = 16
N_TILES = N_CORES * N_SUBCORES
LANES = 16

CAND_PER_TILE = 6400
GATHER_W = 128
N_GATHERS = CAND_PER_TILE // GATHER_W
N_ITERS = CAND_PER_TILE // (2 * LANES)
HALF = CAND_PER_TILE // 2

TAB_SLICE = 3136
TAB_PAD = N_SUBCORES * TAB_SLICE


def _ln(x):
    xi = plsc.bitcast(x, jnp.int32)
    e = ((xi >> 23) & 0xFF) - 127
    m = plsc.bitcast((xi & 0x7FFFFF) | 0x3F800000, jnp.float32)
    big = m > 1.4142135
    m = jnp.where(big, m * 0.5, m)
    ef = e.astype(jnp.float32) + jnp.where(big, 1.0, 0.0)
    t = (m - 1.0) / (m + 1.0)
    t2 = t * t
    p = t * (2.0 + t2 * (0.6666667 + t2 * (0.4 + t2 * (0.2857143 + t2 * 0.22222223))))
    return ef * 0.6931472 + p


def _make_sc_kernel(n_cands):
    mesh = plsc.VectorSubcoreMesh(core_axis_name="c", subcore_axis_name="s")

    @functools.partial(
        pl.kernel,
        out_type=jax.ShapeDtypeStruct((N_TILES, LANES), jnp.float32),
        mesh=mesh,
        scratch_types=[
            pltpu.VMEM((TAB_PAD,), jnp.float32),
            pltpu.VMEM((TAB_PAD,), jnp.float32),
            pltpu.VMEM((TAB_SLICE,), jnp.float32),
            pltpu.VMEM((TAB_SLICE,), jnp.float32),
            pltpu.VMEM((HALF,), jnp.int32),
            pltpu.VMEM((CAND_PER_TILE,), jnp.int32),
            pltpu.VMEM((CAND_PER_TILE,), jnp.int32),
            pltpu.VMEM((LANES,), jnp.float32),
            pltpu.VMEM_SHARED((TAB_PAD,), jnp.float32),
            pltpu.VMEM_SHARED((TAB_PAD,), jnp.float32),
            pltpu.SemaphoreType.DMA,
            pltpu.SemaphoreType.DMA,
        ],
        compiler_params=pltpu.CompilerParams(
            needs_layout_passes=False, use_tc_tiling_on_sc=False),
    )
    def sck(restx_hbm, resty_hbm, ux_hbm, uy_hbm, epk_hbm, cvp_hbm, ce_hbm,
            out_hbm, currx_v, curry_v, ubufx_v, ubufy_v, cv_v, ce_v, epk_v,
            acc_v, currx_sh, curry_sh, sem_g, sem_s):
        c = lax.axis_index("c")
        s = lax.axis_index("s")
        wid = s * N_CORES + c
        myoff = s * TAB_SLICE
        sl = pl.ds(myoff, TAB_SLICE)

        with jax.named_scope("stage"):
            stage = [
                pltpu.make_async_copy(cvp_hbm.at[wid], cv_v, sem_s),
                pltpu.make_async_copy(ce_hbm.at[wid], ce_v, sem_s),
                pltpu.make_async_copy(restx_hbm.at[sl], currx_v.at[sl], sem_s),
                pltpu.make_async_copy(resty_hbm.at[sl], curry_v.at[sl], sem_s),
                pltpu.make_async_copy(ux_hbm.at[sl], ubufx_v, sem_s),
                pltpu.make_async_copy(uy_hbm.at[sl], ubufy_v, sem_s),
            ]
            for d in stage:
                d.start()
            for d in stage:
                d.wait()

        with jax.named_scope("fire_gathers"):
            @pl.loop(0, N_GATHERS)
            def _(k):
                pltpu.make_async_copy(
                    epk_hbm.at[ce_v.at[pl.ds(k * GATHER_W, GATHER_W)]],
                    epk_v.at[pl.ds(k * GATHER_W, GATHER_W)], sem_g).start()

        with jax.named_scope("build_slices"):
            @pl.loop(0, TAB_SLICE // LANES)
            def _(j):
                d = pl.ds(myoff + j * LANES, LANES)
                b = pl.ds(j * LANES, LANES)
                currx_v[d] = currx_v[d] + ubufx_v[b]
                curry_v[d] = curry_v[d] + ubufy_v[b]

            pltpu.sync_copy(currx_v.at[sl], currx_sh.at[sl])
            pltpu.sync_copy(curry_v.at[sl], curry_sh.at[sl])

        with jax.named_scope("barrier"):
            plsc.subcore_barrier()
        with jax.named_scope("broadcast"):
            pltpu.sync_copy(currx_sh, currx_v)
            pltpu.sync_copy(curry_sh, curry_v)

        with jax.named_scope("drain_gathers"):
            @pl.loop(0, N_GATHERS)
            def _(k):
                pltpu.make_async_copy(
                    epk_hbm.at[ce_v.at[pl.ds(k * GATHER_W, GATHER_W)]],
                    epk_v.at[pl.ds(k * GATHER_W, GATHER_W)], sem_g).wait()

        lanes = lax.iota(jnp.int32, LANES)
        base_g = wid * CAND_PER_TILE

        def contrib(cv, pk, g):
            e0i = pk & 0xFFFF
            e1i = (pk >> 16) & 0xFFFF
            px = plsc.load_gather(currx_v, [cv])
            py = plsc.load_gather(curry_v, [cv])
            e0x = plsc.load_gather(currx_v, [e0i])
            e0y = plsc.load_gather(curry_v, [e0i])
            e1x = plsc.load_gather(currx_v, [e1i])
            e1y = plsc.load_gather(curry_v, [e1i])

            dex = e1x - e0x
            dey = e1y - e0y
            dd = jnp.maximum(dex * dex + dey * dey, 1e-12)
            qx = px - e0x
            qy = py - e0y
            t = (qx * dex + qy * dey) / dd
            t = jnp.minimum(jnp.maximum(t, 0.0), 1.0)
            cx = e0x + t * dex
            cy = e0y + t * dey
            dx = px - cx
            dy = py - cy
            d2 = dx * dx + dy * dy

            active = (d2 < DHAT2) & (d2 > 0.0)
            d2s = jnp.where(active, d2, DHAT2)
            diff = d2s - DHAT2
            b = -(diff * diff) * _ln(d2s * INV_DHAT2)
            return jnp.where(active & (g < n_cands), b, 0.0)

        with jax.named_scope("mainloop"):
            @plsc.parallel_loop(0, N_ITERS, unroll=2,
                                carry=jnp.zeros((LANES,), jnp.float32))
            def acc(j, acc_in):
                jb = j * LANES
                cvp = cv_v[pl.ds(jb, LANES)]
                pk_lo = epk_v[pl.ds(jb, LANES)]
                pk_hi = epk_v[pl.ds(HALF + jb, LANES)]
                b_lo = contrib(cvp & 0xFFFF, pk_lo, base_g + jb + lanes)
                b_hi = contrib((cvp >> 16) & 0xFFFF, pk_hi,
                               base_g + HALF + jb + lanes)
                return acc_in + b_lo + b_hi

            acc_v[...] = acc
        pltpu.sync_copy(acc_v, out_hbm.at[wid])

    return sck


def kernel(U, rest, edges, cand_v, cand_e):
    n_verts = rest.shape[0]
    n_cands = cand_v.shape[0]
    pad_v = TAB_PAD - n_verts
    restx = jnp.pad(rest[:, 0], (0, pad_v))
    resty = jnp.pad(rest[:, 1], (0, pad_v))
    ux = jnp.pad(U[:, 0], (0, pad_v))
    uy = jnp.pad(U[:, 1], (0, pad_v))
    epk = edges[:, 0] | (edges[:, 1] << 16)
    pad_c = N_TILES * CAND_PER_TILE - n_cands
    cv2 = jnp.pad(cand_v, (0, pad_c)).reshape(N_TILES, 2, HALF)
    cvp = cv2[:, 0] | (cv2[:, 1] << 16)
    ce = jnp.pad(cand_e, (0, pad_c)).reshape(N_TILES, CAND_PER_TILE)
    out = _make_sc_kernel(n_cands)(restx, resty, ux, uy, epk, cvp, ce)
    return jnp.sum(out)
